# baseline (speedup 1.0000x reference)
import functools

import jax
import jax.numpy as jnp
from jax import lax
from jax.experimental import pallas as pl
from jax.experimental.pallas import tpu as pltpu
from jax.experimental.pallas import tpu_sc as plsc

_NC = 2
_NS = 16
_C = 128
_G = 10


def _make_sc_segment_sum(N, D, tch, nch0, nch1):
    G = _G
    rpt = N // _NS
    mesh = plsc.VectorSubcoreMesh(core_axis_name="c", subcore_axis_name="s")

    @functools.partial(
        pl.kernel,
        mesh=mesh,
        out_type=jax.ShapeDtypeStruct((_NC, N, D), jnp.float32),
        scratch_types=[
            pltpu.VMEM_SHARED((N, D), jnp.float32),
            pltpu.VMEM((G, 2, 128), jnp.int32),
            pltpu.VMEM((G, 2, 128), jnp.int32),
            pltpu.VMEM((G * _C,), jnp.float32),
            pltpu.VMEM((G * _C,), jnp.float32),
            pltpu.VMEM((_C, D), jnp.float32),
            pltpu.VMEM((_C, D), jnp.float32),
            pltpu.SemaphoreType.DMA,
            pltpu.SemaphoreType.DMA,
        ],
    )
    def sc_seg_sum(ego, pk, wh, out, acc, idxA, idxB, wA, wB, rows0, rows1,
                   g0, g1):
        c = lax.axis_index("c")
        s = lax.axis_index("s")
        pk_base = jnp.where(c == 0, s * nch0, _NS * nch0 + s * nch1)
        n_outer = jnp.where(c == 0, nch0 // (2 * G), nch1 // (2 * G))
        zero16 = jnp.zeros((16,), jnp.float32)
        rows = (rows0, rows1)
        gsem = (g0, g1)

        def scale(w_v, k, rows_v):
            dnums = lax.GatherDimensionNumbers(
                offset_dims=(), collapsed_slice_dims=(0,),
                start_index_map=(0,))

            def body(ib, carry):
                wvec = w_v[pl.ds(k * _C + ib * 16, 16)]
                for lane in range(16):
                    wb = lax.gather(
                        wvec, jnp.full((16, 1), lane, jnp.int32), dnums, (1,),
                        mode=lax.GatherScatterMode.PROMISE_IN_BOUNDS)
                    i = ib * 16 + lane
                    for r in range(D // 16):
                        sl = pl.ds(r * 16, 16)
                        rows_v[i, sl] = rows_v[i, sl] * wb
                return carry

            lax.fori_loop(0, _C // 16, body, 0)

        def load_group(idx_v, w_v, group):
            gchunk = pk_base + group * G
            pltpu.sync_copy(pk.at[pl.ds(gchunk, G)], idx_v)
            pltpu.sync_copy(wh.at[pl.ds(gchunk * _C, G * _C)], w_v)

        def gather(idx_v, k, rows_v, sem):
            pltpu.async_copy(ego.at[idx_v.at[k, 0]], rows_v, sem)

        def wait_gather(idx_v, k, rows_v, sem):
            pltpu.make_async_copy(ego.at[idx_v.at[k, 0]], rows_v, sem).wait()

        def scatter(idx_v, k, rows_v):
            pltpu.sync_copy(rows_v, acc.at[idx_v.at[k, 1]], add=True)

        def zbody(i, carry):
            for r in range(D // 16):
                rows0[i, pl.ds(r * 16, 16)] = zero16
            return carry

        lax.fori_loop(0, _C, zbody, 0)
        base = s * rpt
        left = rpt
        off = 0
        while left > 0:
            step = min(left, _C)
            pltpu.sync_copy(rows0.at[pl.ds(0, step)],
                            acc.at[pl.ds(base + off, step)])
            off += step
            left -= step
        plsc.subcore_barrier()

        def prologue():
            load_group(idxA, wA, 0)
            gather(idxA, 0, rows0, g0)

        pl.when(n_outer > 0)(prologue)

        def process(idx_v, w_v, k, gpar):
            p = k % 2
            wait_gather(idx_v, k, rows[p], gsem[p])
            if gpar == 0:
                nidx, nk = (idx_v, k + 1) if k < G - 1 else (idxB, 0)
                gather(nidx, nk, rows[1 - p], gsem[1 - p])
            else:
                if k < G - 1:
                    gather(idx_v, k + 1, rows[1 - p], gsem[1 - p])
            scale(w_v, k, rows[p])
            scatter(idx_v, k, rows[p])

        def outer(gi, carry):
            load_group(idxB, wB, 2 * gi + 1)
            for k in range(G - 1):
                process(idxA, wA, k, 0)
            process(idxA, wA, G - 1, 0)
            pl.when(gi < n_outer - 1)(
                lambda: load_group(idxA, wA, 2 * gi + 2))
            for k in range(G - 1):
                process(idxB, wB, k, 1)
            p = (G - 1) % 2
            wait_gather(idxB, G - 1, rows[p], gsem[p])
            pl.when(gi < n_outer - 1)(
                lambda: gather(idxA, 0, rows[1 - p], gsem[1 - p]))
            scale(wB, G - 1, rows[p])
            scatter(idxB, G - 1, rows[p])
            return carry

        with jax.named_scope("zz_edges"):
            lax.fori_loop(0, n_outer, outer, 0)
            plsc.subcore_barrier()
        with jax.named_scope("zz_copyout"):
            pltpu.sync_copy(acc.at[pl.ds(base, rpt)],
                            out.at[c, pl.ds(base, rpt)])

    return sc_seg_sum


def _tc_dense(ego, p0, p1, W, b, g, beta):
    N, D = ego.shape
    BM = N // 8
    grid = (N // BM,)

    def body(ego_ref, p0_ref, p1_ref, w_ref, b_ref, g_ref, bt_ref, o_ref):
        hi = ego_ref[...] + p0_ref[...] + p1_ref[...]
        e = lax.dot_general(hi, w_ref[...], (((1,), (1,)), ((), ())),
                            preferred_element_type=jnp.float32)
        e = e + b_ref[...]
        e = jnp.where(e >= 0, e, e * 0.01)
        mu = jnp.mean(e, axis=-1, keepdims=True)
        var = jnp.mean((e - mu) ** 2, axis=-1, keepdims=True)
        o_ref[...] = (e - mu) * lax.rsqrt(var + 1e-5) * g_ref[...] + bt_ref[...]

    return pl.pallas_call(
        body,
        grid=grid,
        in_specs=[
            pl.BlockSpec((BM, D), lambda i: (i, 0)),
            pl.BlockSpec((BM, D), lambda i: (i, 0)),
            pl.BlockSpec((BM, D), lambda i: (i, 0)),
            pl.BlockSpec((D, D), lambda i: (0, 0)),
            pl.BlockSpec((1, D), lambda i: (0, 0)),
            pl.BlockSpec((1, D), lambda i: (0, 0)),
            pl.BlockSpec((1, D), lambda i: (0, 0)),
        ],
        out_specs=pl.BlockSpec((BM, D), lambda i: (i, 0)),
        out_shape=jax.ShapeDtypeStruct((N, D), jnp.float32),
    )(ego, p0, p1, W, b.reshape(1, D), g.reshape(1, D), beta.reshape(1, D))


def kernel(ego_embeddings, edge_index, edge_weight, W_lin, b_lin, ln_gamma, ln_beta):
    N, D = ego_embeddings.shape
    E = edge_weight.shape[0]
    NW = _NC * _NS
    ept = -(-E // (NW * 2 * _G * _C)) * (2 * _G * _C)
    E_pad = ept * NW
    tch = E_pad // _C
    pair = tch // _NS
    blk = 2 * _G
    nch0 = max(blk, min(pair - blk, (int(round(pair * 0.75)) // blk) * blk))
    nch1 = pair - nch0
    nch0, nch1 = pair, 0
    pad = E_pad - E
    src = edge_index[0]
    dst = edge_index[1]
    w = edge_weight
    if pad:
        src = jnp.concatenate([src, jnp.zeros((pad,), src.dtype)])
        dst = jnp.concatenate([dst, jnp.zeros((pad,), dst.dtype)])
        w = jnp.concatenate([w, jnp.zeros((pad,), w.dtype)])
    srcm = src.reshape(E_pad // _C, _C)
    dstm = dst.reshape(E_pad // _C, _C)
    pk = jnp.stack([srcm, dstm], axis=1)

    N_pad = -(-N // (8 * _NS)) * (8 * _NS)
    sc = _make_sc_segment_sum(N_pad, D, tch, nch0, nch1)
    partials = sc(ego_embeddings, pk, w)
    ego_p = ego_embeddings
    if N_pad != N:
        ego_p = jnp.concatenate(
            [ego_embeddings, jnp.zeros((N_pad - N, D), ego_embeddings.dtype)])
    out = _tc_dense(ego_p, partials[0], partials[1],
                    W_lin, b_lin, ln_gamma, ln_beta)
    return out[:N]


# speedup vs baseline: 3.8150x; 3.8150x over previous
import functools

import jax
import jax.numpy as jnp
from jax import lax
from jax.experimental import pallas as pl
from jax.experimental.pallas import tpu as pltpu
from jax.experimental.pallas import tpu_sc as plsc

_NC = 2
_NS = 16
_C = 128
_G = 10


def _make_sc_segment_sum(N, D, tch, nch0, nch1):
    G = _G
    rpt = N // _NS
    mesh = plsc.VectorSubcoreMesh(core_axis_name="c", subcore_axis_name="s")

    @functools.partial(
        pl.kernel,
        mesh=mesh,
        out_type=jax.ShapeDtypeStruct((_NC, N, D), jnp.float32),
        scratch_types=[
            pltpu.VMEM_SHARED((N, D), jnp.float32),
            pltpu.VMEM((G, 2, 128), jnp.int32),
            pltpu.VMEM((G, 2, 128), jnp.int32),
            pltpu.VMEM((G * _C,), jnp.float32),
            pltpu.VMEM((G * _C,), jnp.float32),
            pltpu.VMEM((_C, D), jnp.float32),
            pltpu.VMEM((_C, D), jnp.float32),
            pltpu.SemaphoreType.DMA,
            pltpu.SemaphoreType.DMA,
        ],
    )
    def sc_seg_sum(ego, pk, wh, out, acc, idxA, idxB, wA, wB, rows0, rows1,
                   g0, g1):
        c = lax.axis_index("c")
        s = lax.axis_index("s")
        pk_base = jnp.where(c == 0, s * nch0, _NS * nch0 + s * nch1)
        n_outer = jnp.where(c == 0, nch0 // (2 * G), nch1 // (2 * G))
        zero16 = jnp.zeros((16,), jnp.float32)
        rows = (rows0, rows1)
        gsem = (g0, g1)

        def scale(w_v, k, rows_v):
            dnums = lax.GatherDimensionNumbers(
                offset_dims=(), collapsed_slice_dims=(0,),
                start_index_map=(0,))

            def body(ib, carry):
                wvec = w_v[pl.ds(k * _C + ib * 16, 16)]
                for lane in range(16):
                    wb = lax.gather(
                        wvec, jnp.full((16, 1), lane, jnp.int32), dnums, (1,),
                        mode=lax.GatherScatterMode.PROMISE_IN_BOUNDS)
                    i = ib * 16 + lane
                    for r in range(D // 16):
                        sl = pl.ds(r * 16, 16)
                        rows_v[i, sl] = rows_v[i, sl] * wb
                return carry

            lax.fori_loop(0, _C // 16, body, 0)

        def load_group(idx_v, w_v, group):
            gchunk = pk_base + group * G
            pltpu.sync_copy(pk.at[pl.ds(gchunk, G)], idx_v)
            pltpu.sync_copy(wh.at[pl.ds(gchunk * _C, G * _C)], w_v)

        def gather(idx_v, k, rows_v, sem):
            pltpu.async_copy(ego.at[idx_v.at[k, 0]], rows_v, sem)

        def wait_gather(idx_v, k, rows_v, sem):
            pltpu.make_async_copy(ego.at[idx_v.at[k, 0]], rows_v, sem).wait()

        def scatter(idx_v, k, rows_v):
            pltpu.sync_copy(rows_v, acc.at[idx_v.at[k, 1]], add=True)

        def zbody(i, carry):
            for r in range(D // 16):
                rows0[i, pl.ds(r * 16, 16)] = zero16
            return carry

        lax.fori_loop(0, _C, zbody, 0)
        base = s * rpt
        left = rpt
        off = 0
        while left > 0:
            step = min(left, _C)
            pltpu.sync_copy(rows0.at[pl.ds(0, step)],
                            acc.at[pl.ds(base + off, step)])
            off += step
            left -= step
        plsc.subcore_barrier()

        def prologue():
            load_group(idxA, wA, 0)
            gather(idxA, 0, rows0, g0)

        pl.when(n_outer > 0)(prologue)

        def process(idx_v, w_v, k, gpar):
            p = k % 2
            wait_gather(idx_v, k, rows[p], gsem[p])
            if gpar == 0:
                nidx, nk = (idx_v, k + 1) if k < G - 1 else (idxB, 0)
                gather(nidx, nk, rows[1 - p], gsem[1 - p])
            else:
                if k < G - 1:
                    gather(idx_v, k + 1, rows[1 - p], gsem[1 - p])
            scale(w_v, k, rows[p])
            scatter(idx_v, k, rows[p])

        def outer(gi, carry):
            load_group(idxB, wB, 2 * gi + 1)
            for k in range(G - 1):
                process(idxA, wA, k, 0)
            process(idxA, wA, G - 1, 0)
            pl.when(gi < n_outer - 1)(
                lambda: load_group(idxA, wA, 2 * gi + 2))
            for k in range(G - 1):
                process(idxB, wB, k, 1)
            p = (G - 1) % 2
            wait_gather(idxB, G - 1, rows[p], gsem[p])
            pl.when(gi < n_outer - 1)(
                lambda: gather(idxA, 0, rows[1 - p], gsem[1 - p]))
            scale(wB, G - 1, rows[p])
            scatter(idxB, G - 1, rows[p])
            return carry

        with jax.named_scope("zz_edges"):
            lax.fori_loop(0, n_outer, outer, 0)
            plsc.subcore_barrier()
        with jax.named_scope("zz_copyout"):
            pltpu.sync_copy(acc.at[pl.ds(base, rpt)],
                            out.at[c, pl.ds(base, rpt)])

    return sc_seg_sum


def _tc_dense(ego, p0, p1, W, b, g, beta):
    N, D = ego.shape
    BM = N // 8
    grid = (N // BM,)

    def body(ego_ref, p0_ref, p1_ref, w_ref, b_ref, g_ref, bt_ref, o_ref):
        hi = ego_ref[...] + p0_ref[...] + p1_ref[...]
        e = lax.dot_general(hi, w_ref[...], (((1,), (1,)), ((), ())),
                            preferred_element_type=jnp.float32)
        e = e + b_ref[...]
        e = jnp.where(e >= 0, e, e * 0.01)
        mu = jnp.mean(e, axis=-1, keepdims=True)
        var = jnp.mean((e - mu) ** 2, axis=-1, keepdims=True)
        o_ref[...] = (e - mu) * lax.rsqrt(var + 1e-5) * g_ref[...] + bt_ref[...]

    return pl.pallas_call(
        body,
        grid=grid,
        in_specs=[
            pl.BlockSpec((BM, D), lambda i: (i, 0)),
            pl.BlockSpec((BM, D), lambda i: (i, 0)),
            pl.BlockSpec((BM, D), lambda i: (i, 0)),
            pl.BlockSpec((D, D), lambda i: (0, 0)),
            pl.BlockSpec((1, D), lambda i: (0, 0)),
            pl.BlockSpec((1, D), lambda i: (0, 0)),
            pl.BlockSpec((1, D), lambda i: (0, 0)),
        ],
        out_specs=pl.BlockSpec((BM, D), lambda i: (i, 0)),
        out_shape=jax.ShapeDtypeStruct((N, D), jnp.float32),
    )(ego, p0, p1, W, b.reshape(1, D), g.reshape(1, D), beta.reshape(1, D))


def kernel(ego_embeddings, edge_index, edge_weight, W_lin, b_lin, ln_gamma, ln_beta):
    N, D = ego_embeddings.shape
    E = edge_weight.shape[0]
    NW = _NC * _NS
    ept = -(-E // (NW * 2 * _G * _C)) * (2 * _G * _C)
    E_pad = ept * NW
    tch = E_pad // _C
    pair = tch // _NS
    blk = 2 * _G
    nch0 = (pair // 2 // blk) * blk
    nch1 = pair - nch0
    pad = E_pad - E
    src = edge_index[0]
    dst = edge_index[1]
    w = edge_weight
    if pad:
        spread = jnp.arange(pad, dtype=src.dtype) % N
        src = jnp.concatenate([src, spread])
        dst = jnp.concatenate([dst, spread])
        w = jnp.concatenate([w, jnp.zeros((pad,), w.dtype)])
    srcm = src.reshape(E_pad // _C, _C)
    dstm = dst.reshape(E_pad // _C, _C)
    pk = jnp.stack([srcm, dstm], axis=1)

    N_pad = -(-N // (8 * _NS)) * (8 * _NS)
    sc = _make_sc_segment_sum(N_pad, D, tch, nch0, nch1)
    partials = sc(ego_embeddings, pk, w)
    ego_p = ego_embeddings
    if N_pad != N:
        ego_p = jnp.concatenate(
            [ego_embeddings, jnp.zeros((N_pad - N, D), ego_embeddings.dtype)])
    out = _tc_dense(ego_p, partials[0], partials[1],
                    W_lin, b_lin, ln_gamma, ln_beta)
    return out[:N]


# speedup vs baseline: 3.8666x; 1.0135x over previous
import functools

import jax
import jax.numpy as jnp
from jax import lax
from jax.experimental import pallas as pl
from jax.experimental.pallas import tpu as pltpu
from jax.experimental.pallas import tpu_sc as plsc

_NC = 2
_NS = 16
_C = 128
_G = 10


def _make_sc_segment_sum(N, D, tch, nch0, nch1):
    G = _G
    rpt = N // _NS
    mesh = plsc.VectorSubcoreMesh(core_axis_name="c", subcore_axis_name="s")

    @functools.partial(
        pl.kernel,
        mesh=mesh,
        out_type=jax.ShapeDtypeStruct((_NC, N, D), jnp.float32),
        scratch_types=[
            pltpu.VMEM_SHARED((N, D), jnp.float32),
            pltpu.VMEM((G, 2, 128), jnp.int32),
            pltpu.VMEM((G, 2, 128), jnp.int32),
            pltpu.VMEM((G * _C,), jnp.float32),
            pltpu.VMEM((G * _C,), jnp.float32),
            pltpu.VMEM((_C, D), jnp.float32),
            pltpu.VMEM((_C, D), jnp.float32),
            pltpu.SemaphoreType.DMA,
            pltpu.SemaphoreType.DMA,
            pltpu.SemaphoreType.DMA,
            pltpu.SemaphoreType.DMA,
        ],
    )
    def sc_seg_sum(ego, pk, wh, out, acc, idxA, idxB, wA, wB, rows0, rows1,
                   g0, g1, s0, s1):
        c = lax.axis_index("c")
        s = lax.axis_index("s")
        pk_base = jnp.where(c == 0, s * nch0, _NS * nch0 + s * nch1)
        n_outer = jnp.where(c == 0, nch0 // (2 * G), nch1 // (2 * G))
        zero16 = jnp.zeros((16,), jnp.float32)
        rows = (rows0, rows1)
        gsem = (g0, g1)
        ssem = (s0, s1)

        def scale(w_v, k, rows_v):
            dnums = lax.GatherDimensionNumbers(
                offset_dims=(), collapsed_slice_dims=(0,),
                start_index_map=(0,))

            def body(ib, carry):
                wvec = w_v[pl.ds(k * _C + ib * 16, 16)]
                for lane in range(16):
                    wb = lax.gather(
                        wvec, jnp.full((16, 1), lane, jnp.int32), dnums, (1,),
                        mode=lax.GatherScatterMode.PROMISE_IN_BOUNDS)
                    i = ib * 16 + lane
                    for r in range(D // 16):
                        sl = pl.ds(r * 16, 16)
                        rows_v[i, sl] = rows_v[i, sl] * wb
                return carry

            lax.fori_loop(0, _C // 16, body, 0)

        def load_group(idx_v, w_v, group):
            gchunk = pk_base + group * G
            pltpu.sync_copy(pk.at[pl.ds(gchunk, G)], idx_v)
            pltpu.sync_copy(wh.at[pl.ds(gchunk * _C, G * _C)], w_v)

        def gather(idx_v, k, rows_v, sem):
            pltpu.async_copy(ego.at[idx_v.at[k, 0]], rows_v, sem)

        def wait_gather(idx_v, k, rows_v, sem):
            pltpu.make_async_copy(ego.at[idx_v.at[k, 0]], rows_v, sem).wait()

        def scatter(idx_v, k, rows_v, sem):
            pltpu.async_copy(rows_v, acc.at[idx_v.at[k, 1]], sem, add=True)

        def wait_scatter(idx_v, k, rows_v, sem):
            pltpu.make_async_copy(rows_v, acc.at[idx_v.at[k, 1]], sem).wait()

        def zbody(i, carry):
            for r in range(D // 16):
                rows0[i, pl.ds(r * 16, 16)] = zero16
            return carry

        lax.fori_loop(0, _C, zbody, 0)
        base = s * rpt
        left = rpt
        off = 0
        while left > 0:
            step = min(left, _C)
            pltpu.sync_copy(rows0.at[pl.ds(0, step)],
                            acc.at[pl.ds(base + off, step)])
            off += step
            left -= step
        plsc.subcore_barrier()

        def prologue():
            load_group(idxA, wA, 0)
            gather(idxA, 0, rows0, g0)

        pl.when(n_outer > 0)(prologue)

        def process(idx_v, w_v, k, gpar, gi):
            p = k % 2
            wait_gather(idx_v, k, rows[p], gsem[p])
            if not (gpar == 0 and k == 0):
                wait_scatter(idx_v, k, rows[1 - p], ssem[1 - p])
            if gpar == 0:
                nidx, nk = (idx_v, k + 1) if k < G - 1 else (idxB, 0)
                gather(nidx, nk, rows[1 - p], gsem[1 - p])
            else:
                if k < G - 1:
                    gather(idx_v, k + 1, rows[1 - p], gsem[1 - p])
            scale(w_v, k, rows[p])
            scatter(idx_v, k, rows[p], ssem[p])

        def outer(gi, carry):
            pl.when(gi > 0)(
                lambda: wait_scatter(idxB, G - 1, rows[(G - 1) % 2],
                                     ssem[(G - 1) % 2]))
            load_group(idxB, wB, 2 * gi + 1)
            for k in range(G):
                process(idxA, wA, k, 0, gi)
            process(idxB, wB, 0, 1, gi)
            pl.when(gi < n_outer - 1)(
                lambda: load_group(idxA, wA, 2 * gi + 2))
            for k in range(1, G - 1):
                process(idxB, wB, k, 1, gi)
            p = (G - 1) % 2
            wait_gather(idxB, G - 1, rows[p], gsem[p])
            wait_scatter(idxB, G - 1, rows[1 - p], ssem[1 - p])
            pl.when(gi < n_outer - 1)(
                lambda: gather(idxA, 0, rows[1 - p], gsem[1 - p]))
            scale(wB, G - 1, rows[p])
            scatter(idxB, G - 1, rows[p], ssem[p])
            return carry

        with jax.named_scope("zz_edges"):
            lax.fori_loop(0, n_outer, outer, 0)
            pl.when(n_outer > 0)(
                lambda: wait_scatter(idxB, G - 1, rows[(G - 1) % 2],
                                     ssem[(G - 1) % 2]))
            plsc.subcore_barrier()
        with jax.named_scope("zz_copyout"):
            pltpu.sync_copy(acc.at[pl.ds(base, rpt)],
                            out.at[c, pl.ds(base, rpt)])

    return sc_seg_sum


def _tc_dense(ego, p0, p1, W, b, g, beta):
    N, D = ego.shape
    BM = N // 8
    grid = (N // BM,)

    def body(ego_ref, p0_ref, p1_ref, w_ref, b_ref, g_ref, bt_ref, o_ref):
        hi = ego_ref[...] + p0_ref[...] + p1_ref[...]
        e = lax.dot_general(hi, w_ref[...], (((1,), (1,)), ((), ())),
                            preferred_element_type=jnp.float32)
        e = e + b_ref[...]
        e = jnp.where(e >= 0, e, e * 0.01)
        mu = jnp.mean(e, axis=-1, keepdims=True)
        var = jnp.mean((e - mu) ** 2, axis=-1, keepdims=True)
        o_ref[...] = (e - mu) * lax.rsqrt(var + 1e-5) * g_ref[...] + bt_ref[...]

    return pl.pallas_call(
        body,
        grid=grid,
        in_specs=[
            pl.BlockSpec((BM, D), lambda i: (i, 0)),
            pl.BlockSpec((BM, D), lambda i: (i, 0)),
            pl.BlockSpec((BM, D), lambda i: (i, 0)),
            pl.BlockSpec((D, D), lambda i: (0, 0)),
            pl.BlockSpec((1, D), lambda i: (0, 0)),
            pl.BlockSpec((1, D), lambda i: (0, 0)),
            pl.BlockSpec((1, D), lambda i: (0, 0)),
        ],
        out_specs=pl.BlockSpec((BM, D), lambda i: (i, 0)),
        out_shape=jax.ShapeDtypeStruct((N, D), jnp.float32),
    )(ego, p0, p1, W, b.reshape(1, D), g.reshape(1, D), beta.reshape(1, D))


def kernel(ego_embeddings, edge_index, edge_weight, W_lin, b_lin, ln_gamma, ln_beta):
    N, D = ego_embeddings.shape
    E = edge_weight.shape[0]
    NW = _NC * _NS
    ept = -(-E // (NW * 2 * _G * _C)) * (2 * _G * _C)
    E_pad = ept * NW
    tch = E_pad // _C
    pair = tch // _NS
    blk = 2 * _G
    nch0 = (pair // 2 // blk) * blk
    nch1 = pair - nch0
    pad = E_pad - E
    src = edge_index[0]
    dst = edge_index[1]
    w = edge_weight
    if pad:
        spread = jnp.arange(pad, dtype=src.dtype) % N
        src = jnp.concatenate([src, spread])
        dst = jnp.concatenate([dst, spread])
        w = jnp.concatenate([w, jnp.zeros((pad,), w.dtype)])
    srcm = src.reshape(E_pad // _C, _C)
    dstm = dst.reshape(E_pad // _C, _C)
    pk = jnp.stack([srcm, dstm], axis=1)

    N_pad = -(-N // (8 * _NS)) * (8 * _NS)
    sc = _make_sc_segment_sum(N_pad, D, tch, nch0, nch1)
    partials = sc(ego_embeddings, pk, w)
    ego_p = ego_embeddings
    if N_pad != N:
        ego_p = jnp.concatenate(
            [ego_embeddings, jnp.zeros((N_pad - N, D), ego_embeddings.dtype)])
    out = _tc_dense(ego_p, partials[0], partials[1],
                    W_lin, b_lin, ln_gamma, ln_beta)
    return out[:N]


# speedup vs baseline: 3.9246x; 1.0150x over previous
import functools

import jax
import jax.numpy as jnp
from jax import lax
from jax.experimental import pallas as pl
from jax.experimental.pallas import tpu as pltpu
from jax.experimental.pallas import tpu_sc as plsc

_NC = 2
_NS = 16
_C = 128
_G = 8


def _make_sc_segment_sum(N, D, tch, nch0, nch1):
    G = _G
    rpt = N // _NS
    mesh = plsc.VectorSubcoreMesh(core_axis_name="c", subcore_axis_name="s")

    @functools.partial(
        pl.kernel,
        mesh=mesh,
        out_type=jax.ShapeDtypeStruct((_NC, N, D), jnp.float32),
        scratch_types=[
            pltpu.VMEM_SHARED((N, D), jnp.float32),
            pltpu.VMEM((G, 128), jnp.int32),
            pltpu.VMEM((G, 128), jnp.int32),
            pltpu.VMEM((G, 128), jnp.int32),
            pltpu.VMEM((G, 128), jnp.int32),
            pltpu.VMEM((G * _C,), jnp.float32),
            pltpu.VMEM((G * _C,), jnp.float32),
            pltpu.VMEM((_C, D), jnp.float32),
            pltpu.VMEM((_C, D), jnp.float32),
            pltpu.SemaphoreType.DMA,
            pltpu.SemaphoreType.DMA,
            pltpu.SemaphoreType.DMA,
            pltpu.SemaphoreType.DMA,
        ],
    )
    def sc_seg_sum(ego, srcm, dstm, wh, out, acc, srcA, srcB, dstA, dstB,
                   wA, wB, rows0, rows1, g0, g1, s0, s1):
        c = lax.axis_index("c")
        s = lax.axis_index("s")
        pk_base = jnp.where(c == 0, s * nch0, _NS * nch0 + s * nch1)
        n_outer = jnp.where(c == 0, nch0 // (2 * G), nch1 // (2 * G))
        zero16 = jnp.zeros((16,), jnp.float32)
        rows = (rows0, rows1)
        gsem = (g0, g1)
        ssem = (s0, s1)

        def scale(w_v, k, rows_v):
            dnums = lax.GatherDimensionNumbers(
                offset_dims=(), collapsed_slice_dims=(0,),
                start_index_map=(0,))

            def body(ib, carry):
                wvec = w_v[pl.ds(k * _C + ib * 16, 16)]
                for lane in range(16):
                    wb = lax.gather(
                        wvec, jnp.full((16, 1), lane, jnp.int32), dnums, (1,),
                        mode=lax.GatherScatterMode.PROMISE_IN_BOUNDS)
                    i = ib * 16 + lane
                    for r in range(D // 16):
                        sl = pl.ds(r * 16, 16)
                        rows_v[i, sl] = rows_v[i, sl] * wb
                return carry

            lax.fori_loop(0, _C // 16, body, 0)

        def load_group(src_v, dst_v, w_v, group):
            gchunk = pk_base + group * G
            pltpu.sync_copy(srcm.at[pl.ds(gchunk, G)], src_v)
            pltpu.sync_copy(dstm.at[pl.ds(gchunk, G)], dst_v)
            pltpu.sync_copy(wh.at[pl.ds(gchunk * _C, G * _C)], w_v)

        def gather(src_v, k, rows_v, sem):
            pltpu.async_copy(ego.at[src_v.at[k]], rows_v, sem)

        def wait_gather(src_v, k, rows_v, sem):
            pltpu.make_async_copy(ego.at[src_v.at[k]], rows_v, sem).wait()

        def scatter(dst_v, k, rows_v, sem):
            pltpu.async_copy(rows_v, acc.at[dst_v.at[k]], sem, add=True)

        def wait_scatter(dst_v, k, rows_v, sem):
            pltpu.make_async_copy(rows_v, acc.at[dst_v.at[k]], sem).wait()

        def zbody(i, carry):
            for r in range(D // 16):
                rows0[i, pl.ds(r * 16, 16)] = zero16
            return carry

        lax.fori_loop(0, _C, zbody, 0)
        base = s * rpt
        left = rpt
        off = 0
        while left > 0:
            step = min(left, _C)
            pltpu.sync_copy(rows0.at[pl.ds(0, step)],
                            acc.at[pl.ds(base + off, step)])
            off += step
            left -= step
        plsc.subcore_barrier()

        def prologue():
            load_group(srcA, dstA, wA, 0)
            gather(srcA, 0, rows0, g0)

        pl.when(n_outer > 0)(prologue)

        def process(src_v, dst_v, w_v, k, gpar, gi):
            p = k % 2
            wait_gather(src_v, k, rows[p], gsem[p])
            if not (gpar == 0 and k == 0):
                wait_scatter(dst_v, k, rows[1 - p], ssem[1 - p])
            if gpar == 0:
                nsrc, nk = (src_v, k + 1) if k < G - 1 else (srcB, 0)
                gather(nsrc, nk, rows[1 - p], gsem[1 - p])
            else:
                if k < G - 1:
                    gather(src_v, k + 1, rows[1 - p], gsem[1 - p])
            scale(w_v, k, rows[p])
            scatter(dst_v, k, rows[p], ssem[p])

        def outer(gi, carry):
            pl.when(gi > 0)(
                lambda: wait_scatter(dstB, G - 1, rows[(G - 1) % 2],
                                     ssem[(G - 1) % 2]))
            load_group(srcB, dstB, wB, 2 * gi + 1)
            for k in range(G):
                process(srcA, dstA, wA, k, 0, gi)
            process(srcB, dstB, wB, 0, 1, gi)
            pl.when(gi < n_outer - 1)(
                lambda: load_group(srcA, dstA, wA, 2 * gi + 2))
            for k in range(1, G - 1):
                process(srcB, dstB, wB, k, 1, gi)
            p = (G - 1) % 2
            wait_gather(srcB, G - 1, rows[p], gsem[p])
            wait_scatter(dstB, G - 1, rows[1 - p], ssem[1 - p])
            pl.when(gi < n_outer - 1)(
                lambda: gather(srcA, 0, rows[1 - p], gsem[1 - p]))
            scale(wB, G - 1, rows[p])
            scatter(dstB, G - 1, rows[p], ssem[p])
            return carry

        with jax.named_scope("zz_edges"):
            lax.fori_loop(0, n_outer, outer, 0)
            pl.when(n_outer > 0)(
                lambda: wait_scatter(dstB, G - 1, rows[(G - 1) % 2],
                                     ssem[(G - 1) % 2]))
            plsc.subcore_barrier()
        with jax.named_scope("zz_copyout"):
            pltpu.sync_copy(acc.at[pl.ds(base, rpt)],
                            out.at[c, pl.ds(base, rpt)])

    return sc_seg_sum


def _tc_dense(ego, p0, p1, W, b, g, beta):
    N, D = ego.shape
    BM = next(b for b in (2000, 1000, 500, 200, 40, 8)
              if N % b == 0)
    grid = (N // BM,)

    def body(ego_ref, p0_ref, p1_ref, w_ref, b_ref, g_ref, bt_ref, o_ref):
        hi = ego_ref[...] + p0_ref[...] + p1_ref[...]
        e = lax.dot_general(hi, w_ref[...], (((1,), (1,)), ((), ())),
                            preferred_element_type=jnp.float32)
        e = e + b_ref[...]
        e = jnp.where(e >= 0, e, e * 0.01)
        mu = jnp.mean(e, axis=-1, keepdims=True)
        var = jnp.mean((e - mu) ** 2, axis=-1, keepdims=True)
        o_ref[...] = (e - mu) * lax.rsqrt(var + 1e-5) * g_ref[...] + bt_ref[...]

    return pl.pallas_call(
        body,
        grid=grid,
        in_specs=[
            pl.BlockSpec((BM, D), lambda i: (i, 0)),
            pl.BlockSpec((BM, D), lambda i: (i, 0)),
            pl.BlockSpec((BM, D), lambda i: (i, 0)),
            pl.BlockSpec((D, D), lambda i: (0, 0)),
            pl.BlockSpec((1, D), lambda i: (0, 0)),
            pl.BlockSpec((1, D), lambda i: (0, 0)),
            pl.BlockSpec((1, D), lambda i: (0, 0)),
        ],
        out_specs=pl.BlockSpec((BM, D), lambda i: (i, 0)),
        out_shape=jax.ShapeDtypeStruct((N, D), jnp.float32),
    )(ego, p0, p1, W, b.reshape(1, D), g.reshape(1, D), beta.reshape(1, D))


def kernel(ego_embeddings, edge_index, edge_weight, W_lin, b_lin, ln_gamma, ln_beta):
    N, D = ego_embeddings.shape
    E = edge_weight.shape[0]
    NW = _NC * _NS
    ept = -(-E // (NW * 2 * _G * _C)) * (2 * _G * _C)
    E_pad = ept * NW
    tch = E_pad // _C
    pair = tch // _NS
    blk = 2 * _G
    nch0 = (pair // 2 // blk) * blk
    nch1 = pair - nch0
    pad = E_pad - E
    src = edge_index[0]
    dst = edge_index[1]
    w = edge_weight
    if pad:
        spread = jnp.arange(pad, dtype=src.dtype) % N
        src = jnp.concatenate([src, spread])
        dst = jnp.concatenate([dst, spread])
        w = jnp.concatenate([w, jnp.zeros((pad,), w.dtype)])
    srcm = src.reshape(E_pad // _C, _C)
    dstm = dst.reshape(E_pad // _C, _C)

    N_pad = -(-N // (8 * _NS)) * (8 * _NS)
    sc = _make_sc_segment_sum(N_pad, D, tch, nch0, nch1)
    partials = sc(ego_embeddings, srcm, dstm, w)
    return _tc_dense(ego_embeddings, partials[0], partials[1],
                     W_lin, b_lin, ln_gamma, ln_beta)
